# baseline (speedup 1.0000x reference)
import jax
import jax.numpy as jnp
from jax import lax
from jax.experimental import pallas as pl
from jax.experimental.pallas import tpu as pltpu
from jax.experimental.pallas import tpu_sc as plsc

_E = 6_400_000
_N = 100_000
_NC, _NS = 2, 16
_NW = _NC * _NS
_NPAD = 102_400
_SLICE = _NPAD // _NS
_CHUNK = _NPAD // _NW
_UNIT = 128
_NU = _E // _UNIT
_BASE_U = _NU // _NW
_EXTRA = _NU - _BASE_U * _NW
_CU = 64
_FULL = (_BASE_U // _CU)
_TAIL_U = _BASE_U - _FULL * _CU
_ECHUNK = _CU * _UNIT
_ETAIL = _TAIL_U * _UNIT

_mesh = plsc.VectorSubcoreMesh(core_axis_name="c", subcore_axis_name="s")


def _destride_row1(ib, ic, n):
    def _cp(k, carry):
        base = k * 128
        for j in range(8):
            ic[pl.ds(base + j * 16, 16)] = ib[1, pl.ds(base + j * 16, 16)]
        return carry

    lax.fori_loop(0, n // 128, _cp, 0)


def _scatter_body(eij_hbm, ea_hbm, part_hbm, idx0, val0, idx1, val1,
                  idx2, val2, ic0, ic1, ic2,
                  idx_t, val_t, ic_t, idx_x, val_x, ic_x, acc_sh, cp_v,
                  ld_sem0, ld_sem1, ld_sem2, sc_sem0, sc_sem1, sc_sem2):
    c = lax.axis_index("c")
    s = lax.axis_index("s")
    wid = s * _NC + c

    e0 = (wid * _BASE_U + jnp.minimum(wid, _EXTRA)) * _UNIT
    bufs = ((idx0, val0, ic0, ld_sem0, sc_sem0),
            (idx1, val1, ic1, ld_sem1, sc_sem1),
            (idx2, val2, ic2, ld_sem2, sc_sem2))

    def fire_loads(i):
        ib, vb, _, ls, _ = bufs[i % 3]
        off = e0 + i * _ECHUNK
        d1 = pltpu.async_copy(eij_hbm.at[:, pl.ds(off, _ECHUNK)], ib, ls)
        d2 = pltpu.async_copy(ea_hbm.at[pl.ds(off, _ECHUNK)], vb, ls)
        return d1, d2

    descs = {0: fire_loads(0), 1: fire_loads(1)}

    def _z(j, carry):
        cp_v[pl.ds(j * 16, 16)] = jnp.zeros((16,), jnp.float32)
        return carry

    lax.fori_loop(0, _SLICE // 16, _z, 0)
    pltpu.sync_copy(cp_v, acc_sh.at[pl.ds(s * _SLICE, _SLICE)])
    plsc.subcore_barrier()

    sc_prev = None
    for i in range(_FULL):
        ib, vb, ic, _, ss = bufs[i % 3]
        d = descs.pop(i)
        for dd in d:
            dd.wait()
        _destride_row1(ib, ic, _ECHUNK)
        if sc_prev is not None:
            sc_prev.wait()
        if i + 2 < _FULL:
            descs[i + 2] = fire_loads(i + 2)
        sc_prev = pltpu.async_copy(vb, acc_sh.at[ic], ss, add=True)
    sc_prev.wait()

    off = e0 + _FULL * _ECHUNK
    pltpu.sync_copy(eij_hbm.at[:, pl.ds(off, _ETAIL)], idx_t)
    _destride_row1(idx_t, ic_t, _ETAIL)
    pltpu.sync_copy(ea_hbm.at[pl.ds(off, _ETAIL)], val_t)
    pltpu.sync_copy(val_t, acc_sh.at[ic_t], add=True)

    @pl.when(wid < _EXTRA)
    def _extra_unit():
        off2 = e0 + _FULL * _ECHUNK + _ETAIL
        pltpu.sync_copy(eij_hbm.at[:, pl.ds(off2, _UNIT)], idx_x)
        _destride_row1(idx_x, ic_x, _UNIT)
        pltpu.sync_copy(ea_hbm.at[pl.ds(off2, _UNIT)], val_x)
        pltpu.sync_copy(val_x, acc_sh.at[ic_x], add=True)

    plsc.subcore_barrier()
    pltpu.sync_copy(acc_sh.at[pl.ds(s * _SLICE, _SLICE)],
                    part_hbm.at[pl.ds(c * _NPAD + s * _SLICE, _SLICE)])


def _combine_tc(part_ref, out_ref):
    _half = _NPAD // 128
    out_ref[...] = part_ref[0:_half, :] + part_ref[_half:2 * _half, :]


_k1 = pl.kernel(
    _scatter_body,
    mesh=_mesh,
    out_type=jax.ShapeDtypeStruct((_NC * _NPAD,), jnp.float32),
    scratch_types=[
        pltpu.VMEM((2, _ECHUNK), jnp.int32),
        pltpu.VMEM((_ECHUNK,), jnp.float32),
        pltpu.VMEM((2, _ECHUNK), jnp.int32),
        pltpu.VMEM((_ECHUNK,), jnp.float32),
        pltpu.VMEM((2, _ECHUNK), jnp.int32),
        pltpu.VMEM((_ECHUNK,), jnp.float32),
        pltpu.VMEM((_ECHUNK,), jnp.int32),
        pltpu.VMEM((_ECHUNK,), jnp.int32),
        pltpu.VMEM((_ECHUNK,), jnp.int32),
        pltpu.VMEM((2, _ETAIL), jnp.int32),
        pltpu.VMEM((_ETAIL,), jnp.float32),
        pltpu.VMEM((_ETAIL,), jnp.int32),
        pltpu.VMEM((2, _UNIT), jnp.int32),
        pltpu.VMEM((_UNIT,), jnp.float32),
        pltpu.VMEM((_UNIT,), jnp.int32),
        pltpu.VMEM_SHARED((_NPAD,), jnp.float32),
        pltpu.VMEM((_SLICE,), jnp.float32),
        pltpu.SemaphoreType.DMA,
        pltpu.SemaphoreType.DMA,
        pltpu.SemaphoreType.DMA,
        pltpu.SemaphoreType.DMA,
        pltpu.SemaphoreType.DMA,
        pltpu.SemaphoreType.DMA,
    ],
)

_k2 = pl.pallas_call(
    _combine_tc,
    out_shape=jax.ShapeDtypeStruct((_NPAD // 128, 128), jnp.float32),
)


def kernel(vertex_attr, edgeij_pair, edge_attr, g, batch):
    ea = edge_attr.reshape(_E)
    part = _k1(edgeij_pair, ea)
    b = _k2(part.reshape(_NC * _NPAD // 128, 128))
    return jnp.concatenate([b.reshape(_NPAD)[:_N, None], vertex_attr[:, 1:2]],
                           axis=1)


# speedup vs baseline: 1.0091x; 1.0091x over previous
import jax
import jax.numpy as jnp
from jax import lax
from jax.experimental import pallas as pl
from jax.experimental.pallas import tpu as pltpu
from jax.experimental.pallas import tpu_sc as plsc

_E = 6_400_000
_N = 100_000
_NC, _NS = 2, 16
_NW = _NC * _NS
_NPAD = 102_400
_SLICE = _NPAD // _NS
_CHUNK = _NPAD // _NW
_UNIT = 128
_NU = _E // _UNIT
_BASE_U = _NU // _NW
_EXTRA = _NU - _BASE_U * _NW
_CU = 48
_FULL = (_BASE_U // _CU)
_TAIL_U = _BASE_U - _FULL * _CU
_ECHUNK = _CU * _UNIT
_ETAIL = _TAIL_U * _UNIT

_mesh = plsc.VectorSubcoreMesh(core_axis_name="c", subcore_axis_name="s")


def _destride_row1(ib, ic, n):
    def _cp(k, carry):
        base = k * 128
        for j in range(8):
            ic[pl.ds(base + j * 16, 16)] = ib[1, pl.ds(base + j * 16, 16)]
        return carry

    lax.fori_loop(0, n // 128, _cp, 0)


def _scatter_body(eij_hbm, ea_hbm, part_hbm, idx0, val0, idx1, val1,
                  idx2, val2, idx3, val3, ic0, ic1, ic2, ic3,
                  idx_t, val_t, ic_t, idx_x, val_x, ic_x, acc_sh, cp_v,
                  ld_sem0, ld_sem1, ld_sem2, ld_sem3,
                  sc_sem0, sc_sem1, sc_sem2, sc_sem3):
    c = lax.axis_index("c")
    s = lax.axis_index("s")
    wid = s * _NC + c

    e0 = (wid * _BASE_U + jnp.minimum(wid, _EXTRA)) * _UNIT
    bufs = ((idx0, val0, ic0, ld_sem0, sc_sem0),
            (idx1, val1, ic1, ld_sem1, sc_sem1),
            (idx2, val2, ic2, ld_sem2, sc_sem2),
            (idx3, val3, ic3, ld_sem3, sc_sem3))

    def fire_loads(i):
        ib, vb, _, ls, _ = bufs[i % 4]
        off = e0 + i * _ECHUNK
        d1 = pltpu.async_copy(eij_hbm.at[:, pl.ds(off, _ECHUNK)], ib, ls)
        d2 = pltpu.async_copy(ea_hbm.at[pl.ds(off, _ECHUNK)], vb, ls)
        return d1, d2

    descs = {0: fire_loads(0), 1: fire_loads(1)}

    def _z(j, carry):
        cp_v[pl.ds(j * 16, 16)] = jnp.zeros((16,), jnp.float32)
        return carry

    lax.fori_loop(0, _SLICE // 16, _z, 0)
    pltpu.sync_copy(cp_v, acc_sh.at[pl.ds(s * _SLICE, _SLICE)])
    plsc.subcore_barrier()

    scd = {}
    for i in range(_FULL):
        ib, vb, ic, _, ss = bufs[i % 4]
        d = descs.pop(i)
        for dd in d:
            dd.wait()
        _destride_row1(ib, ic, _ECHUNK)
        if i >= 2:
            scd.pop(i - 2).wait()
        if i + 2 < _FULL:
            descs[i + 2] = fire_loads(i + 2)
        scd[i] = pltpu.async_copy(vb, acc_sh.at[ic], ss, add=True)
    for dd in scd.values():
        dd.wait()

    off = e0 + _FULL * _ECHUNK
    pltpu.sync_copy(eij_hbm.at[:, pl.ds(off, _ETAIL)], idx_t)
    _destride_row1(idx_t, ic_t, _ETAIL)
    pltpu.sync_copy(ea_hbm.at[pl.ds(off, _ETAIL)], val_t)
    pltpu.sync_copy(val_t, acc_sh.at[ic_t], add=True)

    @pl.when(wid < _EXTRA)
    def _extra_unit():
        off2 = e0 + _FULL * _ECHUNK + _ETAIL
        pltpu.sync_copy(eij_hbm.at[:, pl.ds(off2, _UNIT)], idx_x)
        _destride_row1(idx_x, ic_x, _UNIT)
        pltpu.sync_copy(ea_hbm.at[pl.ds(off2, _UNIT)], val_x)
        pltpu.sync_copy(val_x, acc_sh.at[ic_x], add=True)

    plsc.subcore_barrier()
    pltpu.sync_copy(acc_sh.at[pl.ds(s * _SLICE, _SLICE)],
                    part_hbm.at[pl.ds(c * _NPAD + s * _SLICE, _SLICE)])


def _combine_tc(part_ref, out_ref):
    _half = _NPAD // 128
    out_ref[...] = part_ref[0:_half, :] + part_ref[_half:2 * _half, :]


_k1 = pl.kernel(
    _scatter_body,
    mesh=_mesh,
    out_type=jax.ShapeDtypeStruct((_NC * _NPAD,), jnp.float32),
    scratch_types=[
        pltpu.VMEM((2, _ECHUNK), jnp.int32),
        pltpu.VMEM((_ECHUNK,), jnp.float32),
        pltpu.VMEM((2, _ECHUNK), jnp.int32),
        pltpu.VMEM((_ECHUNK,), jnp.float32),
        pltpu.VMEM((2, _ECHUNK), jnp.int32),
        pltpu.VMEM((_ECHUNK,), jnp.float32),
        pltpu.VMEM((2, _ECHUNK), jnp.int32),
        pltpu.VMEM((_ECHUNK,), jnp.float32),
        pltpu.VMEM((_ECHUNK,), jnp.int32),
        pltpu.VMEM((_ECHUNK,), jnp.int32),
        pltpu.VMEM((_ECHUNK,), jnp.int32),
        pltpu.VMEM((_ECHUNK,), jnp.int32),
        pltpu.VMEM((2, _ETAIL), jnp.int32),
        pltpu.VMEM((_ETAIL,), jnp.float32),
        pltpu.VMEM((_ETAIL,), jnp.int32),
        pltpu.VMEM((2, _UNIT), jnp.int32),
        pltpu.VMEM((_UNIT,), jnp.float32),
        pltpu.VMEM((_UNIT,), jnp.int32),
        pltpu.VMEM_SHARED((_NPAD,), jnp.float32),
        pltpu.VMEM((_SLICE,), jnp.float32),
        pltpu.SemaphoreType.DMA,
        pltpu.SemaphoreType.DMA,
        pltpu.SemaphoreType.DMA,
        pltpu.SemaphoreType.DMA,
        pltpu.SemaphoreType.DMA,
        pltpu.SemaphoreType.DMA,
        pltpu.SemaphoreType.DMA,
        pltpu.SemaphoreType.DMA,
    ],
)

_k2 = pl.pallas_call(
    _combine_tc,
    out_shape=jax.ShapeDtypeStruct((_NPAD // 128, 128), jnp.float32),
)


def kernel(vertex_attr, edgeij_pair, edge_attr, g, batch):
    ea = edge_attr.reshape(_E)
    part = _k1(edgeij_pair, ea)
    b = _k2(part.reshape(_NC * _NPAD // 128, 128))
    return jnp.concatenate([b.reshape(_NPAD)[:_N, None], vertex_attr[:, 1:2]],
                           axis=1)
